# baseline (speedup 1.0000x reference)
import functools

import jax
import jax.numpy as jnp
from jax import lax
from jax.experimental import pallas as pl
from jax.experimental.pallas import tpu as pltpu
from jax.experimental.pallas import tpu_sc as plsc

F32 = jnp.float32
HI = lax.Precision.HIGHEST

N = 10000
E = 320000
EP = 327680
H = 128
NG = 64
WIDE = 144

BN = 1000
BE = 2048
BH = 2000
W = 128

@functools.cache
def _vector_mesh():
  return plsc.VectorSubcoreMesh(
      core_axis_name="core", subcore_axis_name="subcore")


def _h0_body(x_ref, atp_ref, pp_ref, btp_ref, o_ref):
  xb = x_ref[...]
  ai = xb[:, 0:1].astype(jnp.int32)
  bi = xb[:, 9:10].astype(jnp.int32)
  ja = lax.broadcasted_iota(jnp.int32, (BN, 128), 1)
  jb = lax.broadcasted_iota(jnp.int32, (BN, 8), 1)
  oa = (ja == ai).astype(F32)
  ob = (jb == bi).astype(F32)
  h = (jnp.dot(oa, atp_ref[...], precision=HI)
       + jnp.dot(xb, pp_ref[...], precision=HI)
       + jnp.dot(ob, btp_ref[...], precision=HI))
  o_ref[...] = h


def _build_h0(x, atp, pp, btp):
  return pl.pallas_call(
      _h0_body,
      grid=(N // BN,),
      in_specs=[
          pl.BlockSpec((BN, 10), lambda i: (i, 0)),
          pl.BlockSpec((128, 128), lambda i: (0, 0)),
          pl.BlockSpec((10, 128), lambda i: (0, 0)),
          pl.BlockSpec((8, 128), lambda i: (0, 0)),
      ],
      out_specs=pl.BlockSpec((BN, 128), lambda i: (i, 0)),
      out_shape=jax.ShapeDtypeStruct((N, 128), F32),
  )(x, atp, pp, btp)


def _hist_body(at_ref, o_ref):
  i = pl.program_id(0)
  at = at_ref[...]
  bti = at[:, 0:1].astype(jnp.int32)
  ef = at[:, 1:2]
  cji = at[:, 2:3].astype(jnp.int32)
  ari = at[:, 3:4].astype(jnp.int32)
  j = lax.broadcasted_iota(jnp.int32, (BH, 48), 1)
  f = ((j == bti).astype(F32) + (j == cji + 24).astype(F32)
       + (j == ari + 28).astype(F32))
  f = f + jnp.where(j == 32, ef, 0.0)
  s = jnp.sum(f, axis=0, keepdims=True)

  @pl.when(i == 0)
  def _():
    o_ref[...] = jnp.zeros((1, 48), F32)

  o_ref[...] += s

  @pl.when(i == pl.num_programs(0) - 1)
  def _():
    o_ref[...] = o_ref[...] * (1.0 / E)


def _hist(edge_attr):
  return pl.pallas_call(
      _hist_body,
      grid=(E // BH,),
      in_specs=[pl.BlockSpec((BH, 4), lambda i: (i, 0))],
      out_specs=pl.BlockSpec((1, 48), lambda i: (0, 0)),
      out_shape=jax.ShapeDtypeStruct((1, 48), F32),
  )(edge_attr)


def _xlxr_body(h_ref, wl_ref, bl_ref, wr_ref, br_ref, xl_ref, xr_ref):
  hb = h_ref[...]
  xl_ref[...] = jnp.dot(hb, wl_ref[...], precision=HI) + bl_ref[...]
  xr_ref[...] = jnp.dot(hb, wr_ref[...], precision=HI) + br_ref[...]


def _xlxr(h, wl, bl, wr, br):
  return pl.pallas_call(
      _xlxr_body,
      grid=(N // BN,),
      in_specs=[
          pl.BlockSpec((BN, 128), lambda i: (i, 0)),
          pl.BlockSpec((128, 128), lambda i: (0, 0)),
          pl.BlockSpec((1, 128), lambda i: (0, 0)),
          pl.BlockSpec((128, 128), lambda i: (0, 0)),
          pl.BlockSpec((1, 128), lambda i: (0, 0)),
      ],
      out_specs=[
          pl.BlockSpec((BN, 128), lambda i: (i, 0)),
          pl.BlockSpec((BN, 128), lambda i: (i, 0)),
      ],
      out_shape=[
          jax.ShapeDtypeStruct((N, 128), F32),
          jax.ShapeDtypeStruct((N, 128), F32),
      ],
  )(h, wl, bl, wr, br)


def _edge_body(xs_ref, xd_ref, at_ref, a48_ref, we_ref, att_ref, w_ref,
               e_ref):
  i = pl.program_id(0)
  xs = xs_ref[...]
  xd = xd_ref[...]
  at = at_ref[...]
  bti = at[:, 0:1].astype(jnp.int32)
  ef = at[:, 1:2]
  cji = at[:, 2:3].astype(jnp.int32)
  ari = at[:, 3:4].astype(jnp.int32)
  j = lax.broadcasted_iota(jnp.int32, (BE, 48), 1)
  f = ((j == bti).astype(F32) + (j == cji + 24).astype(F32)
       + (j == ari + 28).astype(F32))
  f = f + jnp.where(j == 32, ef, 0.0)
  t48 = jnp.dot(a48_ref[...], we_ref[...], precision=HI)
  ee = jnp.dot(f, t48, precision=HI)
  m = xs + xd + ee
  m = jnp.where(m > 0, m, 0.2 * m)
  alpha = jnp.sum(m * att_ref[...], axis=1, keepdims=True)
  eidx = i * BE + lax.broadcasted_iota(jnp.int32, (BE, 1), 0)
  ex = jnp.where(eidx < E, jnp.exp(alpha), 0.0)
  w_ref[...] = ex * xs
  e_ref[...] = ex


def _edge_math(xs, xd, attr_p, a48, we, att):
  return pl.pallas_call(
      _edge_body,
      grid=(EP // BE,),
      in_specs=[
          pl.BlockSpec((BE, 128), lambda i: (i, 0)),
          pl.BlockSpec((BE, 128), lambda i: (i, 0)),
          pl.BlockSpec((BE, 4), lambda i: (i, 0)),
          pl.BlockSpec((48, 13), lambda i: (0, 0)),
          pl.BlockSpec((13, 128), lambda i: (0, 0)),
          pl.BlockSpec((1, 128), lambda i: (0, 0)),
      ],
      out_specs=[
          pl.BlockSpec((BE, 128), lambda i: (i, 0)),
          pl.BlockSpec((BE, 1), lambda i: (i, 0)),
      ],
      out_shape=[
          jax.ShapeDtypeStruct((EP, 128), F32),
          jax.ShapeDtypeStruct((EP, 1), F32),
      ],
  )(xs, xd, attr_p, a48, we, att)


def _comb_body(pw_ref, pd_ref, xl_ref, xr_ref, fr_ref, a48_ref, we_ref,
               att_ref, bias_ref, h_ref):
  t48 = jnp.dot(a48_ref[...], we_ref[...], precision=HI)
  eec = jnp.dot(fr_ref[...], t48, precision=HI)
  xl = xl_ref[...]
  xr = xr_ref[...]
  m = xl + xr + eec
  m = jnp.where(m > 0, m, 0.2 * m)
  a_s = jnp.sum(m * att_ref[...], axis=1, keepdims=True)
  ex_s = jnp.exp(a_s)
  out_u = pw_ref[0] + pw_ref[1] + ex_s * xl
  den = jnp.sum(pd_ref[...], axis=0) + ex_s
  h = out_u / (den + 1e-16) + bias_ref[...]
  h_ref[...] = jnp.maximum(h, 0.0)


def _combine(pw, pd3, xl, xr, freq48, a48, we, att, bias):
  return pl.pallas_call(
      _comb_body,
      grid=(N // BN,),
      in_specs=[
          pl.BlockSpec((2, BN, 128), lambda i: (0, i, 0)),
          pl.BlockSpec((32, BN, 1), lambda i: (0, i, 0)),
          pl.BlockSpec((BN, 128), lambda i: (i, 0)),
          pl.BlockSpec((BN, 128), lambda i: (i, 0)),
          pl.BlockSpec((1, 48), lambda i: (0, 0)),
          pl.BlockSpec((48, 13), lambda i: (0, 0)),
          pl.BlockSpec((13, 128), lambda i: (0, 0)),
          pl.BlockSpec((1, 128), lambda i: (0, 0)),
          pl.BlockSpec((1, 128), lambda i: (0, 0)),
      ],
      out_specs=pl.BlockSpec((BN, 128), lambda i: (i, 0)),
      out_shape=jax.ShapeDtypeStruct((N, 128), F32),
  )(pw, pd3, xl, xr, freq48, a48, we, att, bias)


def _pool_body(h_ref, b_ref, o_ref):
  i = pl.program_id(0)

  @pl.when(i == 0)
  def _():
    o_ref[...] = jnp.full((NG, 128), -jnp.inf, F32)

  hb = h_ref[...]
  bb = b_ref[...]
  for g in range(NG):
    vals = jnp.where(bb == g, hb, -jnp.inf)
    o_ref[g:g + 1, :] = jnp.maximum(
        o_ref[g:g + 1, :], jnp.max(vals, axis=0, keepdims=True))


def _pool(h, batch2d):
  return pl.pallas_call(
      _pool_body,
      grid=(N // BN,),
      in_specs=[
          pl.BlockSpec((BN, 128), lambda i: (i, 0)),
          pl.BlockSpec((BN, 1), lambda i: (i, 0)),
      ],
      out_specs=pl.BlockSpec((NG, 128), lambda i: (0, 0)),
      out_shape=jax.ShapeDtypeStruct((NG, 128), F32),
  )(h, batch2d)


CH = 128
EPW = EP // 32
NCH = EPW // CH


def _sc_gather2(xl, xr, sd):
  @functools.partial(
      pl.kernel,
      out_type=(jax.ShapeDtypeStruct((EP, 128), F32),
                jax.ShapeDtypeStruct((EP, 128), F32)),
      mesh=_vector_mesh(),
      scratch_types=[
          pltpu.VMEM((2, 2, CH), jnp.int32),
          pltpu.VMEM((2, CH, 128), F32),
          pltpu.VMEM((2, CH, 128), F32),
          pltpu.SemaphoreType.DMA((2,)),
          pltpu.SemaphoreType.DMA((2,)),
          pltpu.SemaphoreType.DMA((2,)),
          pltpu.SemaphoreType.DMA((2,)),
          pltpu.SemaphoreType.DMA((2,)),
      ],
  )
  def k(xl_hbm, xr_hbm, sd_hbm, xs_hbm, xd_hbm,
        ib, xsb, xdb, i_sem, gs_sem, gd_sem, ws_sem, wd_sem):
    cid = lax.axis_index("core")
    sid = lax.axis_index("subcore")
    base = (cid * 16 + sid) * EPW

    for b in range(2):
      pltpu.async_copy(sd_hbm.at[:, pl.ds(base + b * CH, CH)], ib.at[b],
                       i_sem.at[b])

    @pl.loop(0, NCH, step=2)
    def _(i0):
      for b in range(2):
        i = i0 + b
        nb = 1 - b
        off = base + i * CH

        @pl.when(i >= 2)
        def _():
          pltpu.make_async_copy(xsb.at[b], xs_hbm.at[pl.ds(off - 2 * CH, CH)],
                                ws_sem.at[b]).wait()
          pltpu.make_async_copy(xdb.at[b], xd_hbm.at[pl.ds(off - 2 * CH, CH)],
                                wd_sem.at[b]).wait()

        pltpu.make_async_copy(sd_hbm.at[:, pl.ds(off, CH)], ib.at[b],
                              i_sem.at[b]).wait()
        pltpu.async_copy(xl_hbm.at[ib.at[b, 0]], xsb.at[b], gs_sem.at[b])
        pltpu.async_copy(xr_hbm.at[ib.at[b, 1]], xdb.at[b], gd_sem.at[b])

        @pl.when(i >= 1)
        def _():
          poff = off - CH
          pltpu.make_async_copy(xl_hbm.at[ib.at[nb, 0]], xsb.at[nb],
                                gs_sem.at[nb]).wait()
          pltpu.make_async_copy(xr_hbm.at[ib.at[nb, 1]], xdb.at[nb],
                                gd_sem.at[nb]).wait()
          pltpu.async_copy(xsb.at[nb], xs_hbm.at[pl.ds(poff, CH)],
                           ws_sem.at[nb])
          pltpu.async_copy(xdb.at[nb], xd_hbm.at[pl.ds(poff, CH)],
                           wd_sem.at[nb])

          @pl.when(i + 1 < NCH)
          def _():
            pltpu.async_copy(sd_hbm.at[:, pl.ds(off + CH, CH)], ib.at[nb],
                             i_sem.at[nb])

    bl = (NCH - 1) % 2
    bl2 = 1 - bl
    end = base + NCH * CH
    pltpu.make_async_copy(xl_hbm.at[ib.at[bl, 0]], xsb.at[bl],
                          gs_sem.at[bl]).wait()
    pltpu.make_async_copy(xr_hbm.at[ib.at[bl, 1]], xdb.at[bl],
                          gd_sem.at[bl]).wait()
    pltpu.async_copy(xsb.at[bl], xs_hbm.at[pl.ds(end - CH, CH)],
                     ws_sem.at[bl])
    pltpu.async_copy(xdb.at[bl], xd_hbm.at[pl.ds(end - CH, CH)],
                     wd_sem.at[bl])
    pltpu.make_async_copy(xsb.at[bl2], xs_hbm.at[pl.ds(end - 2 * CH, CH)],
                          ws_sem.at[bl2]).wait()
    pltpu.make_async_copy(xdb.at[bl2], xd_hbm.at[pl.ds(end - 2 * CH, CH)],
                          wd_sem.at[bl2]).wait()
    pltpu.make_async_copy(xsb.at[bl], xs_hbm.at[pl.ds(end - CH, CH)],
                          ws_sem.at[bl]).wait()
    pltpu.make_async_copy(xdb.at[bl], xd_hbm.at[pl.ds(end - CH, CH)],
                          wd_sem.at[bl]).wait()

  return k(xl, xr, sd)


NACC = 10240
RPS = NACC // 16
ZC = 32


def _sc_scatter(wp, exf, di2):
  @functools.partial(
      pl.kernel,
      out_type=(jax.ShapeDtypeStruct((2, NACC, 128), F32),
                jax.ShapeDtypeStruct((32, N), F32)),
      mesh=_vector_mesh(),
      scratch_types=[
          pltpu.VMEM_SHARED((NACC, 128), F32),
          pltpu.VMEM((ZC, 128), F32),
          pltpu.VMEM((N,), F32),
          pltpu.VMEM((2, CH, 128), F32),
          pltpu.VMEM((2, CH), F32),
          pltpu.VMEM((2, 1, CH), jnp.int32),
          pltpu.SemaphoreType.DMA((2,)),
          pltpu.SemaphoreType.DMA((2,)),
          pltpu.SemaphoreType.DMA((2,)),
      ],
      compiler_params=pltpu.CompilerParams(needs_layout_passes=False),
  )
  def k(w_hbm, ex_hbm, di_hbm, ow_hbm, od_hbm, acc_sh, zbuf, den_v,
        wb, exb, ixb, w_sem, e_sem, i_sem):
    cid = lax.axis_index("core")
    sid = lax.axis_index("subcore")
    tid = cid * 16 + sid
    base = tid * EPW

    for b in range(2):
      off0 = base + b * CH
      pltpu.async_copy(w_hbm.at[pl.ds(off0, CH)], wb.at[b], w_sem.at[b])
      pltpu.async_copy(ex_hbm.at[pl.ds(off0, CH)], exb.at[b], e_sem.at[b])
      pltpu.async_copy(di_hbm.at[:, pl.ds(off0, CH)], ixb.at[b], i_sem.at[b])

    @pl.loop(0, ZC)
    def _(r):
      @pl.loop(0, 128, step=16)
      def _(c):
        zbuf[r, pl.ds(c, 16)] = jnp.zeros((16,), F32)

    @pl.loop(0, N, step=16)
    def _(i):
      den_v[pl.ds(i, 16)] = jnp.zeros((16,), F32)

    @pl.loop(0, RPS, step=ZC)
    def _(r):
      pltpu.sync_copy(zbuf, acc_sh.at[pl.ds(sid * RPS + r, ZC)])

    plsc.subcore_barrier()

    @pl.loop(0, NCH, step=2)
    def _(i0):
      for b in range(2):
        i = i0 + b
        nb = 1 - b
        off = base + i * CH
        pltpu.make_async_copy(w_hbm.at[pl.ds(off, CH)], wb.at[b],
                              w_sem.at[b]).wait()
        pltpu.make_async_copy(ex_hbm.at[pl.ds(off, CH)], exb.at[b],
                              e_sem.at[b]).wait()
        pltpu.make_async_copy(di_hbm.at[:, pl.ds(off, CH)], ixb.at[b],
                              i_sem.at[b]).wait()

        @pl.when(jnp.logical_and(i >= 1, i + 1 < NCH))
        def _():
          noff = off + CH
          pltpu.async_copy(w_hbm.at[pl.ds(noff, CH)], wb.at[nb],
                           w_sem.at[nb])
          pltpu.async_copy(ex_hbm.at[pl.ds(noff, CH)], exb.at[nb],
                           e_sem.at[nb])
          pltpu.async_copy(di_hbm.at[:, pl.ds(noff, CH)], ixb.at[nb],
                           i_sem.at[nb])

        pltpu.sync_copy(wb.at[b], acc_sh.at[ixb.at[b, 0]], add=True)
        for kk in range(CH // 16):
          idx = ixb[b, 0, pl.ds(kk * 16, 16)]
          val = exb[b, pl.ds(kk * 16, 16)]
          plsc.addupdate_scatter(den_v, [idx], val)

    plsc.subcore_barrier()

    @pl.loop(0, RPS, step=ZC)
    def _(r):
      pltpu.sync_copy(acc_sh.at[pl.ds(sid * RPS + r, ZC)],
                      ow_hbm.at[cid, pl.ds(sid * RPS + r, ZC)])

    pltpu.sync_copy(den_v, od_hbm.at[tid])

  return k(wp, exf, di2)


def kernel(x, edge_index, edge_attr, batch, atom_table, bond_table,
           bool_table, Wl1, bl1, Wr1, br1, We1, att1, bias1,
           Wl2, bl2, Wr2, br2, We2, att2, bias2):
  atp = jnp.zeros((128, 128), F32).at[:119, :16].set(atom_table)
  pp = jnp.zeros((10, 128), F32).at[1:9, 16:24].set(jnp.eye(8, dtype=F32))
  btp = jnp.zeros((8, 128), F32).at[:3, 24:26].set(bool_table)
  a48 = (jnp.zeros((48, 13), F32)
         .at[0:22, 0:8].set(bond_table)
         .at[24:27, 9:11].set(bool_table)
         .at[28:31, 11:13].set(bool_table)
         .at[32, 8].set(1.0))
  wl1p = jnp.zeros((128, 128), F32).at[:26].set(Wl1)
  wr1p = jnp.zeros((128, 128), F32).at[:26].set(Wr1)

  pad = EP - E
  sd = jnp.pad(edge_index, ((0, 0), (0, pad)))
  di = sd[1:2]
  attr_p = jnp.pad(edge_attr, ((0, pad), (0, 0)))
  batch2d = batch.reshape(N, 1)

  layers = [
      (wl1p, bl1[None], wr1p, br1[None], We1, att1[None], bias1[None]),
      (Wl2, bl2[None], Wr2, br2[None], We2, att2[None], bias2[None]),
      (Wl2, bl2[None], Wr2, br2[None], We2, att2[None], bias2[None]),
  ]

  h = _build_h0(x, atp, pp, btp)
  freq48 = _hist(edge_attr)

  for (wl, bl, wr, br, we, att, bias) in layers:
    xl, xr = _xlxr(h, wl, bl, wr, br)
    xs, xd = _sc_gather2(xl, xr, sd)
    wp, ex = _edge_math(xs, xd, attr_p, a48, we, att)
    pw, pden = _sc_scatter(wp, ex.reshape(EP), di)
    pd3 = pden.reshape(32, N, 1)
    h = _combine(pw, pd3, xl, xr, freq48, a48, we, att, bias)

  return _pool(h, batch2d)


# speedup vs baseline: 1.0074x; 1.0074x over previous
import functools

import jax
import jax.numpy as jnp
from jax import lax
from jax.experimental import pallas as pl
from jax.experimental.pallas import tpu as pltpu
from jax.experimental.pallas import tpu_sc as plsc

F32 = jnp.float32
HI = lax.Precision.HIGHEST

N = 10000
E = 320000
EP = 327680
H = 128
NG = 64
WIDE = 144

BN = 1000
BE = 2048
BH = 2000
W = 128

@functools.cache
def _vector_mesh():
  return plsc.VectorSubcoreMesh(
      core_axis_name="core", subcore_axis_name="subcore")


def _h0_body(x_ref, atp_ref, pp_ref, btp_ref, o_ref):
  xb = x_ref[...]
  ai = xb[:, 0:1].astype(jnp.int32)
  bi = xb[:, 9:10].astype(jnp.int32)
  ja = lax.broadcasted_iota(jnp.int32, (BN, 128), 1)
  jb = lax.broadcasted_iota(jnp.int32, (BN, 8), 1)
  oa = (ja == ai).astype(F32)
  ob = (jb == bi).astype(F32)
  h = (jnp.dot(oa, atp_ref[...], precision=HI)
       + jnp.dot(xb, pp_ref[...], precision=HI)
       + jnp.dot(ob, btp_ref[...], precision=HI))
  o_ref[...] = h


def _build_h0(x, atp, pp, btp):
  return pl.pallas_call(
      _h0_body,
      grid=(N // BN,),
      in_specs=[
          pl.BlockSpec((BN, 10), lambda i: (i, 0)),
          pl.BlockSpec((128, 128), lambda i: (0, 0)),
          pl.BlockSpec((10, 128), lambda i: (0, 0)),
          pl.BlockSpec((8, 128), lambda i: (0, 0)),
      ],
      out_specs=pl.BlockSpec((BN, 128), lambda i: (i, 0)),
      out_shape=jax.ShapeDtypeStruct((N, 128), F32),
  )(x, atp, pp, btp)


def _hist_body(at_ref, o_ref):
  i = pl.program_id(0)
  at = at_ref[...]
  bti = at[:, 0:1].astype(jnp.int32)
  ef = at[:, 1:2]
  cji = at[:, 2:3].astype(jnp.int32)
  ari = at[:, 3:4].astype(jnp.int32)
  j = lax.broadcasted_iota(jnp.int32, (BH, 48), 1)
  f = ((j == bti).astype(F32) + (j == cji + 24).astype(F32)
       + (j == ari + 28).astype(F32))
  f = f + jnp.where(j == 32, ef, 0.0)
  s = jnp.sum(f, axis=0, keepdims=True)

  @pl.when(i == 0)
  def _():
    o_ref[...] = jnp.zeros((1, 48), F32)

  o_ref[...] += s

  @pl.when(i == pl.num_programs(0) - 1)
  def _():
    o_ref[...] = o_ref[...] * (1.0 / E)


def _hist(edge_attr):
  return pl.pallas_call(
      _hist_body,
      grid=(E // BH,),
      in_specs=[pl.BlockSpec((BH, 4), lambda i: (i, 0))],
      out_specs=pl.BlockSpec((1, 48), lambda i: (0, 0)),
      out_shape=jax.ShapeDtypeStruct((1, 48), F32),
  )(edge_attr)


def _xlxr_body(h_ref, wl_ref, bl_ref, wr_ref, br_ref, xl_ref, xr_ref):
  hb = h_ref[...]
  xl_ref[...] = jnp.dot(hb, wl_ref[...], precision=HI) + bl_ref[...]
  xr_ref[...] = jnp.dot(hb, wr_ref[...], precision=HI) + br_ref[...]


def _xlxr(h, wl, bl, wr, br):
  return pl.pallas_call(
      _xlxr_body,
      grid=(N // BN,),
      in_specs=[
          pl.BlockSpec((BN, 128), lambda i: (i, 0)),
          pl.BlockSpec((128, 128), lambda i: (0, 0)),
          pl.BlockSpec((1, 128), lambda i: (0, 0)),
          pl.BlockSpec((128, 128), lambda i: (0, 0)),
          pl.BlockSpec((1, 128), lambda i: (0, 0)),
      ],
      out_specs=[
          pl.BlockSpec((BN, 128), lambda i: (i, 0)),
          pl.BlockSpec((BN, 128), lambda i: (i, 0)),
      ],
      out_shape=[
          jax.ShapeDtypeStruct((N, 128), F32),
          jax.ShapeDtypeStruct((N, 128), F32),
      ],
  )(h, wl, bl, wr, br)


def _edge_body(xs_ref, xd_ref, at_ref, a48_ref, we_ref, att_ref, w_ref,
               e_ref):
  i = pl.program_id(0)
  xs = xs_ref[...]
  xd = xd_ref[...]
  at = at_ref[...]
  bti = at[:, 0:1].astype(jnp.int32)
  ef = at[:, 1:2]
  cji = at[:, 2:3].astype(jnp.int32)
  ari = at[:, 3:4].astype(jnp.int32)
  j = lax.broadcasted_iota(jnp.int32, (BE, 48), 1)
  f = ((j == bti).astype(F32) + (j == cji + 24).astype(F32)
       + (j == ari + 28).astype(F32))
  f = f + jnp.where(j == 32, ef, 0.0)
  t48 = jnp.dot(a48_ref[...], we_ref[...], precision=HI)
  ee = jnp.dot(f, t48, precision=HI)
  m = xs + xd + ee
  m = jnp.where(m > 0, m, 0.2 * m)
  alpha = jnp.sum(m * att_ref[...], axis=1, keepdims=True)
  eidx = i * BE + lax.broadcasted_iota(jnp.int32, (BE, 1), 0)
  ex = jnp.where(eidx < E, jnp.exp(alpha), 0.0)
  w_ref[...] = ex * xs
  e_ref[...] = ex


def _edge_math(xs, xd, attr_p, a48, we, att):
  return pl.pallas_call(
      _edge_body,
      grid=(EP // BE,),
      in_specs=[
          pl.BlockSpec((BE, 128), lambda i: (i, 0)),
          pl.BlockSpec((BE, 128), lambda i: (i, 0)),
          pl.BlockSpec((BE, 4), lambda i: (i, 0)),
          pl.BlockSpec((48, 13), lambda i: (0, 0)),
          pl.BlockSpec((13, 128), lambda i: (0, 0)),
          pl.BlockSpec((1, 128), lambda i: (0, 0)),
      ],
      out_specs=[
          pl.BlockSpec((BE, 128), lambda i: (i, 0)),
          pl.BlockSpec((BE, 1), lambda i: (i, 0)),
      ],
      out_shape=[
          jax.ShapeDtypeStruct((EP, 128), F32),
          jax.ShapeDtypeStruct((EP, 1), F32),
      ],
  )(xs, xd, attr_p, a48, we, att)


def _comb_body(pw_ref, pd_ref, xl_ref, xr_ref, fr_ref, a48_ref, we_ref,
               att_ref, bias_ref, h_ref):
  t48 = jnp.dot(a48_ref[...], we_ref[...], precision=HI)
  eec = jnp.dot(fr_ref[...], t48, precision=HI)
  xl = xl_ref[...]
  xr = xr_ref[...]
  m = xl + xr + eec
  m = jnp.where(m > 0, m, 0.2 * m)
  a_s = jnp.sum(m * att_ref[...], axis=1, keepdims=True)
  ex_s = jnp.exp(a_s)
  out_u = pw_ref[0] + pw_ref[1] + ex_s * xl
  den = jnp.sum(pd_ref[...], axis=0) + ex_s
  h = out_u / (den + 1e-16) + bias_ref[...]
  h_ref[...] = jnp.maximum(h, 0.0)


def _combine(pw, pd3, xl, xr, freq48, a48, we, att, bias):
  return pl.pallas_call(
      _comb_body,
      grid=(N // BN,),
      in_specs=[
          pl.BlockSpec((2, BN, 128), lambda i: (0, i, 0)),
          pl.BlockSpec((32, BN, 1), lambda i: (0, i, 0)),
          pl.BlockSpec((BN, 128), lambda i: (i, 0)),
          pl.BlockSpec((BN, 128), lambda i: (i, 0)),
          pl.BlockSpec((1, 48), lambda i: (0, 0)),
          pl.BlockSpec((48, 13), lambda i: (0, 0)),
          pl.BlockSpec((13, 128), lambda i: (0, 0)),
          pl.BlockSpec((1, 128), lambda i: (0, 0)),
          pl.BlockSpec((1, 128), lambda i: (0, 0)),
      ],
      out_specs=pl.BlockSpec((BN, 128), lambda i: (i, 0)),
      out_shape=jax.ShapeDtypeStruct((N, 128), F32),
  )(pw, pd3, xl, xr, freq48, a48, we, att, bias)


def _pool_body(h_ref, b_ref, o_ref):
  i = pl.program_id(0)

  @pl.when(i == 0)
  def _():
    o_ref[...] = jnp.full((NG, 128), -jnp.inf, F32)

  hb = h_ref[...]
  bb = b_ref[...]
  for g in range(NG):
    vals = jnp.where(bb == g, hb, -jnp.inf)
    o_ref[g:g + 1, :] = jnp.maximum(
        o_ref[g:g + 1, :], jnp.max(vals, axis=0, keepdims=True))


def _pool(h, batch2d):
  return pl.pallas_call(
      _pool_body,
      grid=(N // BN,),
      in_specs=[
          pl.BlockSpec((BN, 128), lambda i: (i, 0)),
          pl.BlockSpec((BN, 1), lambda i: (i, 0)),
      ],
      out_specs=pl.BlockSpec((NG, 128), lambda i: (0, 0)),
      out_shape=jax.ShapeDtypeStruct((NG, 128), F32),
  )(h, batch2d)


CH = 128
EPW = EP // 32
NCH = EPW // CH


def _sc_gather2(xl, xr, sd):
  @functools.partial(
      pl.kernel,
      out_type=(jax.ShapeDtypeStruct((EP, 128), F32),
                jax.ShapeDtypeStruct((EP, 128), F32)),
      mesh=_vector_mesh(),
      scratch_types=[
          pltpu.VMEM((2, 2, CH), jnp.int32),
          pltpu.VMEM((2, CH, 128), F32),
          pltpu.VMEM((2, CH, 128), F32),
          pltpu.SemaphoreType.DMA((2,)),
          pltpu.SemaphoreType.DMA((2,)),
          pltpu.SemaphoreType.DMA((2,)),
          pltpu.SemaphoreType.DMA((2,)),
          pltpu.SemaphoreType.DMA((2,)),
      ],
  )
  def k(xl_hbm, xr_hbm, sd_hbm, xs_hbm, xd_hbm,
        ib, xsb, xdb, i_sem, gs_sem, gd_sem, ws_sem, wd_sem):
    cid = lax.axis_index("core")
    sid = lax.axis_index("subcore")
    base = (cid * 16 + sid) * EPW

    for b in range(2):
      pltpu.async_copy(sd_hbm.at[:, pl.ds(base + b * CH, CH)], ib.at[b],
                       i_sem.at[b])

    @pl.loop(0, NCH, step=2)
    def _(i0):
      for b in range(2):
        i = i0 + b
        nb = 1 - b
        off = base + i * CH

        @pl.when(i >= 2)
        def _():
          pltpu.make_async_copy(xsb.at[b], xs_hbm.at[pl.ds(off - 2 * CH, CH)],
                                ws_sem.at[b]).wait()
          pltpu.make_async_copy(xdb.at[b], xd_hbm.at[pl.ds(off - 2 * CH, CH)],
                                wd_sem.at[b]).wait()

        pltpu.make_async_copy(sd_hbm.at[:, pl.ds(off, CH)], ib.at[b],
                              i_sem.at[b]).wait()
        pltpu.async_copy(xl_hbm.at[ib.at[b, 0]], xsb.at[b], gs_sem.at[b])
        pltpu.async_copy(xr_hbm.at[ib.at[b, 1]], xdb.at[b], gd_sem.at[b])

        @pl.when(i >= 1)
        def _():
          poff = off - CH
          pltpu.make_async_copy(xl_hbm.at[ib.at[nb, 0]], xsb.at[nb],
                                gs_sem.at[nb]).wait()
          pltpu.make_async_copy(xr_hbm.at[ib.at[nb, 1]], xdb.at[nb],
                                gd_sem.at[nb]).wait()
          pltpu.async_copy(xsb.at[nb], xs_hbm.at[pl.ds(poff, CH)],
                           ws_sem.at[nb])
          pltpu.async_copy(xdb.at[nb], xd_hbm.at[pl.ds(poff, CH)],
                           wd_sem.at[nb])

          @pl.when(i + 1 < NCH)
          def _():
            pltpu.async_copy(sd_hbm.at[:, pl.ds(off + CH, CH)], ib.at[nb],
                             i_sem.at[nb])

    bl = (NCH - 1) % 2
    bl2 = 1 - bl
    end = base + NCH * CH
    pltpu.make_async_copy(xl_hbm.at[ib.at[bl, 0]], xsb.at[bl],
                          gs_sem.at[bl]).wait()
    pltpu.make_async_copy(xr_hbm.at[ib.at[bl, 1]], xdb.at[bl],
                          gd_sem.at[bl]).wait()
    pltpu.async_copy(xsb.at[bl], xs_hbm.at[pl.ds(end - CH, CH)],
                     ws_sem.at[bl])
    pltpu.async_copy(xdb.at[bl], xd_hbm.at[pl.ds(end - CH, CH)],
                     wd_sem.at[bl])
    pltpu.make_async_copy(xsb.at[bl2], xs_hbm.at[pl.ds(end - 2 * CH, CH)],
                          ws_sem.at[bl2]).wait()
    pltpu.make_async_copy(xdb.at[bl2], xd_hbm.at[pl.ds(end - 2 * CH, CH)],
                          wd_sem.at[bl2]).wait()
    pltpu.make_async_copy(xsb.at[bl], xs_hbm.at[pl.ds(end - CH, CH)],
                          ws_sem.at[bl]).wait()
    pltpu.make_async_copy(xdb.at[bl], xd_hbm.at[pl.ds(end - CH, CH)],
                          wd_sem.at[bl]).wait()

  return k(xl, xr, sd)


NACC = 10240
RPS = NACC // 16
ZC = 32


def _sc_scatter(wp, exf, di2):
  @functools.partial(
      pl.kernel,
      out_type=(jax.ShapeDtypeStruct((2, NACC, 128), F32),
                jax.ShapeDtypeStruct((32, N), F32)),
      mesh=_vector_mesh(),
      scratch_types=[
          pltpu.VMEM_SHARED((NACC, 128), F32),
          pltpu.VMEM((ZC, 128), F32),
          pltpu.VMEM((N,), F32),
      ],
      compiler_params=pltpu.CompilerParams(needs_layout_passes=False),
  )
  def k(w_hbm, ex_hbm, di_hbm, ow_hbm, od_hbm, acc_sh, zbuf, den_v):
    cid = lax.axis_index("core")
    sid = lax.axis_index("subcore")
    tid = cid * 16 + sid

    @pl.loop(0, ZC)
    def _(r):
      @pl.loop(0, 128, step=16)
      def _(c):
        zbuf[r, pl.ds(c, 16)] = jnp.zeros((16,), F32)

    @pl.loop(0, N, step=16)
    def _(i):
      den_v[pl.ds(i, 16)] = jnp.zeros((16,), F32)

    @pl.loop(0, RPS, step=ZC)
    def _(r):
      pltpu.sync_copy(zbuf, acc_sh.at[pl.ds(sid * RPS + r, ZC)])

    plsc.subcore_barrier()

    def body(w_v, e_v, i_v):
      pltpu.sync_copy(w_v, acc_sh.at[i_v.at[0]], add=True)
      for kk in range(W // 16):
        idx = i_v[0, pl.ds(kk * 16, 16)]
        val = e_v[pl.ds(kk * 16, 16)]
        plsc.addupdate_scatter(den_v, [idx], val)

    pltpu.emit_pipeline(
        body,
        grid=(EP // W,),
        in_specs=[
            pl.BlockSpec((W, 128), lambda i: (i, 0)),
            pl.BlockSpec((W,), lambda i: (i,)),
            pl.BlockSpec((1, W), lambda i: (0, i)),
        ],
        out_specs=[],
        core_axis_name=("core", "subcore"),
        dimension_semantics=(pltpu.PARALLEL,),
    )(w_hbm, ex_hbm, di_hbm)

    plsc.subcore_barrier()

    @pl.loop(0, RPS, step=ZC)
    def _(r):
      pltpu.sync_copy(acc_sh.at[pl.ds(sid * RPS + r, ZC)],
                      ow_hbm.at[cid, pl.ds(sid * RPS + r, ZC)])

    pltpu.sync_copy(den_v, od_hbm.at[tid])

  return k(wp, exf, di2)


def kernel(x, edge_index, edge_attr, batch, atom_table, bond_table,
           bool_table, Wl1, bl1, Wr1, br1, We1, att1, bias1,
           Wl2, bl2, Wr2, br2, We2, att2, bias2):
  atp = jnp.zeros((128, 128), F32).at[:119, :16].set(atom_table)
  pp = jnp.zeros((10, 128), F32).at[1:9, 16:24].set(jnp.eye(8, dtype=F32))
  btp = jnp.zeros((8, 128), F32).at[:3, 24:26].set(bool_table)
  a48 = (jnp.zeros((48, 13), F32)
         .at[0:22, 0:8].set(bond_table)
         .at[24:27, 9:11].set(bool_table)
         .at[28:31, 11:13].set(bool_table)
         .at[32, 8].set(1.0))
  wl1p = jnp.zeros((128, 128), F32).at[:26].set(Wl1)
  wr1p = jnp.zeros((128, 128), F32).at[:26].set(Wr1)

  pad = EP - E
  sd = jnp.pad(edge_index, ((0, 0), (0, pad)))
  di = sd[1:2]
  attr_p = jnp.pad(edge_attr, ((0, pad), (0, 0)))
  batch2d = batch.reshape(N, 1)

  layers = [
      (wl1p, bl1[None], wr1p, br1[None], We1, att1[None], bias1[None]),
      (Wl2, bl2[None], Wr2, br2[None], We2, att2[None], bias2[None]),
      (Wl2, bl2[None], Wr2, br2[None], We2, att2[None], bias2[None]),
  ]

  h = _build_h0(x, atp, pp, btp)
  freq48 = _hist(edge_attr)

  for (wl, bl, wr, br, we, att, bias) in layers:
    xl, xr = _xlxr(h, wl, bl, wr, br)
    xs, xd = _sc_gather2(xl, xr, sd)
    wp, ex = _edge_math(xs, xd, attr_p, a48, we, att)
    pw, pden = _sc_scatter(wp, ex.reshape(EP), di)
    pd3 = pden.reshape(32, N, 1)
    h = _combine(pw, pd3, xl, xr, freq48, a48, we, att, bias)

  return _pool(h, batch2d)
